# Initial kernel scaffold; baseline (speedup 1.0000x reference)
import functools

import jax
import jax.numpy as jnp
from jax import lax
from jax.experimental import pallas as pl
from jax.experimental.pallas import tpu as pltpu
from jax.experimental.pallas import tpu_sc as plsc

N = 10000
CIN = 128
COUT = 128
KVOL = 27
EPK = 11852

NC = 2
NS = 16
NW = NC * NS
CHUNK = 128
P = KVOL * EPK
CPW = -(-P // (NW * CHUNK))
TPW = CPW * CHUNK
P_PAD = NW * TPW
ROWS_PER_SUB = 632
NPAD = NS * ROWS_PER_SUB
DUMMY = N


def _mm_body(x_ref, w_ref, y_ref):
    y_ref[0] = jnp.dot(x_ref[...], w_ref[0], preferred_element_type=jnp.float32)


def _matmul_all_offsets(x, w):
    return pl.pallas_call(
        _mm_body,
        grid=(KVOL,),
        in_specs=[
            pl.BlockSpec((N, CIN), lambda k: (0, 0)),
            pl.BlockSpec((1, CIN, COUT), lambda k: (k, 0, 0)),
        ],
        out_specs=pl.BlockSpec((1, N, COUT), lambda k: (k, 0, 0)),
        out_shape=jax.ShapeDtypeStruct((KVOL, N, COUT), jnp.float32),
    )(x, w)


def _sc_body(y_hbm, gidx_hbm, oidx_hbm, zeros_hbm, part_hbm,
             idx_g, idx_o, rows, acc, sem):
    c = lax.axis_index("c")
    s = lax.axis_index("s")
    wid = c * NS + s
    pltpu.sync_copy(zeros_hbm.at[pl.ds(s * ROWS_PER_SUB, ROWS_PER_SUB)],
                    acc.at[pl.ds(s * ROWS_PER_SUB, ROWS_PER_SUB)])
    plsc.subcore_barrier()

    base = wid * TPW

    @pl.loop(0, CPW)
    def _(j):
        off = base + j * CHUNK
        pltpu.sync_copy(gidx_hbm.at[pl.ds(off, CHUNK)], idx_g)
        pltpu.sync_copy(oidx_hbm.at[pl.ds(off, CHUNK)], idx_o)
        pltpu.async_copy(y_hbm.at[idx_g], rows, sem).wait()
        pltpu.sync_copy(rows, acc.at[idx_o], add=True)

    plsc.subcore_barrier()
    pltpu.sync_copy(acc.at[pl.ds(s * ROWS_PER_SUB, ROWS_PER_SUB)],
                    part_hbm.at[c].at[pl.ds(s * ROWS_PER_SUB, ROWS_PER_SUB)])


@functools.partial(
    pl.kernel,
    out_type=jax.ShapeDtypeStruct((NC, NPAD, COUT), jnp.float32),
    mesh=plsc.VectorSubcoreMesh(core_axis_name="c", subcore_axis_name="s"),
    scratch_types=[
        pltpu.VMEM((CHUNK,), jnp.int32),
        pltpu.VMEM((CHUNK,), jnp.int32),
        pltpu.VMEM((CHUNK, COUT), jnp.float32),
        pltpu.VMEM_SHARED((NPAD, COUT), jnp.float32),
        pltpu.SemaphoreType.DMA,
    ],
)
def _sc_gather_scatter(y_hbm, gidx_hbm, oidx_hbm, zeros_hbm, part_hbm,
                       idx_g, idx_o, rows, acc, sem):
    _sc_body(y_hbm, gidx_hbm, oidx_hbm, zeros_hbm, part_hbm,
             idx_g, idx_o, rows, acc, sem)


def _add_body(p_ref, b_ref, o_ref):
    o_ref[...] = p_ref[0] + p_ref[1] + b_ref[...]


def _final_add(part, bias):
    return pl.pallas_call(
        _add_body,
        grid=(8,),
        in_specs=[
            pl.BlockSpec((NC, N // 8, COUT), lambda i: (0, i, 0)),
            pl.BlockSpec((1, COUT), lambda i: (0, 0)),
        ],
        out_specs=pl.BlockSpec((N // 8, COUT), lambda i: (i, 0)),
        out_shape=jax.ShapeDtypeStruct((N, COUT), jnp.float32),
    )(part, bias.reshape(1, COUT))


def kernel(x, imap, omap, kernel, bias):
    y = _matmul_all_offsets(x, kernel)
    y2 = y.reshape(KVOL * N, COUT)
    gidx = (imap + (jnp.arange(KVOL, dtype=jnp.int32) * N)[:, None]).reshape(-1)
    gidx = jnp.concatenate(
        [gidx, jnp.zeros((P_PAD - P,), jnp.int32)])
    oidx = jnp.concatenate(
        [omap.reshape(-1), jnp.full((P_PAD - P,), DUMMY, jnp.int32)])
    zeros = jnp.zeros((NPAD, COUT), jnp.float32)
    part = _sc_gather_scatter(y2, gidx, oidx, zeros)
    return _final_add(part, bias)


# speedup vs baseline: 6.9709x; 6.9709x over previous
import functools

import jax
import jax.numpy as jnp
from jax import lax
from jax.experimental import pallas as pl
from jax.experimental.pallas import tpu as pltpu
from jax.experimental.pallas import tpu_sc as plsc

N = 10000
CIN = 128
COUT = 128
KVOL = 27
EPK = 11852

NC = 2
NS = 16
NW = NC * NS
CHUNK = 128
P = KVOL * EPK
CPW = -(-P // (NW * CHUNK))
TPW = CPW * CHUNK
P_PAD = NW * TPW
ROWS_PER_SUB = 632
NPAD = NS * ROWS_PER_SUB
DUMMY = N


def _mm_body(x_ref, w_ref, y_ref):
    y_ref[0] = jnp.dot(x_ref[...], w_ref[0], preferred_element_type=jnp.float32)


def _matmul_all_offsets(x, w):
    return pl.pallas_call(
        _mm_body,
        grid=(KVOL,),
        in_specs=[
            pl.BlockSpec((N, CIN), lambda k: (0, 0)),
            pl.BlockSpec((1, CIN, COUT), lambda k: (k, 0, 0)),
        ],
        out_specs=pl.BlockSpec((1, N, COUT), lambda k: (k, 0, 0)),
        out_shape=jax.ShapeDtypeStruct((KVOL, N, COUT), jnp.float32),
    )(x, w)


def _sc_body(y_hbm, gidx_hbm, oidx_hbm, zeros_hbm, part_hbm,
             idx_g, idx_o, rows, acc, sem):
    c = lax.axis_index("c")
    s = lax.axis_index("s")
    wid = c * NS + s
    pltpu.sync_copy(zeros_hbm.at[pl.ds(s * ROWS_PER_SUB, ROWS_PER_SUB)],
                    acc.at[pl.ds(s * ROWS_PER_SUB, ROWS_PER_SUB)])
    plsc.subcore_barrier()

    base = wid * TPW

    @pl.loop(0, CPW)
    def _(j):
        off = base + j * CHUNK
        pltpu.sync_copy(gidx_hbm.at[pl.ds(off, CHUNK)], idx_g)
        pltpu.sync_copy(oidx_hbm.at[pl.ds(off, CHUNK)], idx_o)
        pltpu.async_copy(y_hbm.at[idx_g], rows, sem).wait()
        pltpu.sync_copy(rows, acc.at[idx_o], add=True)

    plsc.subcore_barrier()
    pltpu.sync_copy(acc.at[pl.ds(s * ROWS_PER_SUB, ROWS_PER_SUB)],
                    part_hbm.at[c].at[pl.ds(s * ROWS_PER_SUB, ROWS_PER_SUB)])


@functools.partial(
    pl.kernel,
    out_type=jax.ShapeDtypeStruct((NC, NPAD, COUT), jnp.float32),
    mesh=plsc.VectorSubcoreMesh(core_axis_name="c", subcore_axis_name="s"),
    scratch_types=[
        pltpu.VMEM((CHUNK,), jnp.int32),
        pltpu.VMEM((CHUNK,), jnp.int32),
        pltpu.VMEM((CHUNK, COUT), jnp.float32),
        pltpu.VMEM_SHARED((NPAD, COUT), jnp.float32),
        pltpu.SemaphoreType.DMA,
    ],
)
def _sc_gather_scatter(y_hbm, gidx_hbm, oidx_hbm, zeros_hbm, part_hbm,
                       idx_g, idx_o, rows, acc, sem):
    _sc_body(y_hbm, gidx_hbm, oidx_hbm, zeros_hbm, part_hbm,
             idx_g, idx_o, rows, acc, sem)


def _add_body(p_ref, b_ref, o_ref):
    o_ref[...] = p_ref[0] + p_ref[1] + b_ref[...]


def _final_add(part, bias):
    return pl.pallas_call(
        _add_body,
        grid=(5,),
        in_specs=[
            pl.BlockSpec((NC, N // 5, COUT), lambda i: (0, i, 0)),
            pl.BlockSpec((1, COUT), lambda i: (0, 0)),
        ],
        out_specs=pl.BlockSpec((N // 5, COUT), lambda i: (i, 0)),
        out_shape=jax.ShapeDtypeStruct((N, COUT), jnp.float32),
    )(part, bias.reshape(1, COUT))


def kernel(x, imap, omap, kernel, bias):
    y = _matmul_all_offsets(x, kernel)
    y2 = y.reshape(KVOL * N, COUT)
    gidx = (imap + (jnp.arange(KVOL, dtype=jnp.int32) * N)[:, None]).reshape(-1)
    gidx = jnp.concatenate(
        [gidx, jnp.zeros((P_PAD - P,), jnp.int32)])
    oidx = jnp.concatenate(
        [omap.reshape(-1), jnp.full((P_PAD - P,), DUMMY, jnp.int32)])
    zeros = jnp.zeros((NPAD, COUT), jnp.float32)
    part = _sc_gather_scatter(y2, gidx, oidx, zeros)
    return _final_add(part, bias)
